# Initial kernel scaffold; baseline (speedup 1.0000x reference)
import jax
import jax.numpy as jnp
from jax.experimental import pallas as pl


def kernel(x, edge_index, batch, W1, b1, W2, b2, W3, b3):
    raise NotImplementedError("write your pallas kernel here")


# speedup vs baseline: 15.0004x; 15.0004x over previous
import functools

import jax
import jax.numpy as jnp
from jax import lax
from jax.experimental import pallas as pl
from jax.experimental.pallas import tpu as pltpu
from jax.experimental.pallas import tpu_sc as plsc

N = 50000
E = 800000
FIN = 33
HID = 64
C = 6
G = 512

FH = 16
NP = 50176
DACC = 51200
NBP = 50176
EPT = E // 32
EPT3 = E // 16
ECH = 1000
E2P = 851968
EPT2 = E2P // 32
ECH2 = 512

f32 = jnp.float32
i32 = jnp.int32

_MESH = plsc.VectorSubcoreMesh(
    core_axis_name="c", subcore_axis_name="s", num_cores=2, num_subcores=16
)
_SC_PARAMS = pltpu.CompilerParams(use_tc_tiling_on_sc=False)


@functools.partial(
    pl.kernel,
    out_type=jax.ShapeDtypeStruct((2 * DACC,), f32),
    mesh=_MESH,
    compiler_params=_SC_PARAMS,
    scratch_types=[
        pltpu.VMEM((ECH,), i32),
        pltpu.VMEM((1568,), i32),
        pltpu.VMEM((1568,), f32),
        pltpu.VMEM_SHARED((DACC,), f32),
    ],
)
def _k1_histograms(dst_hbm, batcho_hbm, ones_hbm, zeros_hbm, out_hbm,
                   idx_e, idx_b, onesv, acc):
    cid = lax.axis_index("c")
    sid = lax.axis_index("s")
    tid = cid * 16 + sid
    pltpu.sync_copy(ones_hbm, onesv)
    pltpu.sync_copy(zeros_hbm, acc.at[pl.ds(sid * 3200, 3200)])
    plsc.subcore_barrier()

    @pl.loop(0, EPT // ECH)
    def _(i):
        base = tid * EPT + i * ECH
        pltpu.sync_copy(dst_hbm.at[pl.ds(base, ECH)], idx_e)
        pltpu.sync_copy(onesv.at[pl.ds(0, ECH)], acc.at[idx_e], add=True)

    pltpu.sync_copy(batcho_hbm.at[pl.ds(tid * 1568, 1568)], idx_b)
    pltpu.sync_copy(onesv, acc.at[idx_b], add=True)
    plsc.subcore_barrier()
    pltpu.sync_copy(
        acc.at[pl.ds(sid * 3200, 3200)],
        out_hbm.at[pl.ds(cid * DACC + sid * 3200, 3200)],
    )


@functools.partial(
    pl.kernel,
    out_type=(jax.ShapeDtypeStruct((2 * NP, FH), f32),
              jax.ShapeDtypeStruct((NP,), f32)),
    mesh=_MESH,
    compiler_params=_SC_PARAMS,
    scratch_types=[
        pltpu.VMEM((ECH,), i32),
        pltpu.VMEM((ECH,), i32),
        pltpu.VMEM((ECH, FH), f32),
        pltpu.VMEM((ECH,), f32),
        pltpu.VMEM_SHARED((NP, FH), f32),
        pltpu.VMEM_SHARED((NP,), f32),
    ],
)
def _k3_segment_sum(va_hbm, vb_hbm, vc_hbm, src_hbm, dst_hbm,
                    zeros_hbm, zeros1_hbm, out_hbm, out1_hbm,
                    idx_s, idx_d, rows, ele, acc, acc1):
    cid = lax.axis_index("c")
    sid = lax.axis_index("s")
    pltpu.sync_copy(zeros_hbm, acc.at[pl.ds(sid * 3136, 3136)])
    pltpu.sync_copy(zeros1_hbm, acc1.at[pl.ds(sid * 3136, 3136)])
    plsc.subcore_barrier()

    @pl.loop(0, EPT3 // ECH)
    def _(i):
        base = sid * EPT3 + i * ECH
        pltpu.sync_copy(src_hbm.at[pl.ds(base, ECH)], idx_s)
        pltpu.sync_copy(dst_hbm.at[pl.ds(base, ECH)], idx_d)

        @pl.when(cid == 0)
        def _():
            pltpu.sync_copy(va_hbm.at[idx_s], rows)

        @pl.when(cid == 1)
        def _():
            pltpu.sync_copy(vb_hbm.at[idx_s], rows)
            pltpu.sync_copy(vc_hbm.at[idx_s], ele)
            pltpu.sync_copy(ele, acc1.at[idx_d], add=True)

        pltpu.sync_copy(rows, acc.at[idx_d], add=True)

    plsc.subcore_barrier()
    pltpu.sync_copy(
        acc.at[pl.ds(sid * 3136, 3136)],
        out_hbm.at[pl.ds(cid * NP + sid * 3136, 3136)],
    )

    @pl.when(cid == 1)
    def _():
        pltpu.sync_copy(
            acc1.at[pl.ds(sid * 3136, 3136)],
            out1_hbm.at[pl.ds(sid * 3136, 3136)],
        )


@functools.partial(
    pl.kernel,
    out_type=jax.ShapeDtypeStruct((32 * G, HID), f32),
    mesh=_MESH,
    compiler_params=_SC_PARAMS,
    scratch_types=[
        pltpu.VMEM((ECH2,), i32),
        pltpu.VMEM((ECH2,), i32),
        pltpu.VMEM((ECH2,), i32),
        pltpu.VMEM((ECH2,), f32),
        pltpu.VMEM((ECH2, HID), f32),
        pltpu.VMEM((G, HID), f32),
    ],
)
def _k5_pool(u_hbm, srcx_hbm, dstx_hbm, dinv_hbm, batchp_hbm, zeros_hbm,
             out_hbm, idx_s, idx_d, gb, wb, rows, z):
    cid = lax.axis_index("c")
    sid = lax.axis_index("s")
    tid = cid * 16 + sid
    pltpu.sync_copy(zeros_hbm, z)

    @pl.loop(0, EPT2 // ECH2)
    def _(i):
        base = tid * EPT2 + i * ECH2
        pltpu.sync_copy(srcx_hbm.at[pl.ds(base, ECH2)], idx_s)
        pltpu.sync_copy(dstx_hbm.at[pl.ds(base, ECH2)], idx_d)
        pltpu.sync_copy(u_hbm.at[idx_s], rows)
        pltpu.sync_copy(dinv_hbm.at[idx_d], wb)
        pltpu.sync_copy(batchp_hbm.at[idx_d], gb)

        @pl.loop(0, ECH2, step=16)
        def _(k):
            gv = gb[pl.ds(k, 16)]
            wv = wb[pl.ds(k, 16)]
            for l in range(16):
                g = gv[l]
                w = wv[l]
                for j in range(HID // 16):
                    sl = pl.ds(j * 16, 16)
                    z[g, sl] = z[g, sl] + rows[k + l, sl] * w

    pltpu.sync_copy(z, out_hbm.at[pl.ds(tid * G, G)])


def _tc_scale_body(deg0, deg1, xa, xb, xc, dinv_o, va_o, vb_o, vc_o):
    d = deg0[...] + deg1[...] + 1.0
    dv = lax.rsqrt(jnp.maximum(d, 1.0))
    dinv_o[...] = dv
    va_o[...] = dv * xa[...]
    vb_o[...] = dv * xb[...]
    vc_o[...] = dv * xc[...]


_tc_scale = pl.pallas_call(
    _tc_scale_body,
    grid=(32,),
    in_specs=[
        pl.BlockSpec((1568, 1), lambda i: (i, 0)),
        pl.BlockSpec((1568, 1), lambda i: (i, 0)),
        pl.BlockSpec((1568, FH), lambda i: (i, 0)),
        pl.BlockSpec((1568, FH), lambda i: (i, 0)),
        pl.BlockSpec((1568, 1), lambda i: (i, 0)),
    ],
    out_specs=[
        pl.BlockSpec((1568, 1), lambda i: (i, 0)),
        pl.BlockSpec((1568, FH), lambda i: (i, 0)),
        pl.BlockSpec((1568, FH), lambda i: (i, 0)),
        pl.BlockSpec((1568, 1), lambda i: (i, 0)),
    ],
    out_shape=[
        jax.ShapeDtypeStruct((NP, 1), f32),
        jax.ShapeDtypeStruct((NP, FH), f32),
        jax.ShapeDtypeStruct((NP, FH), f32),
        jax.ShapeDtypeStruct((NP, 1), f32),
    ],
)


def _tc_hidden_body(sa, sb, sc, va, vb, vc, dinvb, w1a, w1b, w1c, b1, u_o):
    dv = dinvb[...]
    agg_a = dv * (sa[...] + va[...])
    agg_b = dv * (sb[...] + vb[...])
    agg_c = dv * (sc[...] + vc[...])
    h = (jnp.dot(agg_a, w1a[...], preferred_element_type=f32)
         + jnp.dot(agg_b, w1b[...], preferred_element_type=f32)
         + agg_c * w1c[...]
         + b1[...])
    h = jnp.maximum(h, 0.0)
    u = dv * h
    pid = pl.program_id(0)
    rid = pid * 1568 + lax.broadcasted_iota(i32, (1568, 1), 0)
    u_o[...] = jnp.where(rid < N, u, 0.0)


_tc_hidden = pl.pallas_call(
    _tc_hidden_body,
    grid=(32,),
    in_specs=[
        pl.BlockSpec((1568, FH), lambda i: (i, 0)),
        pl.BlockSpec((1568, FH), lambda i: (i, 0)),
        pl.BlockSpec((1568, 1), lambda i: (i, 0)),
        pl.BlockSpec((1568, FH), lambda i: (i, 0)),
        pl.BlockSpec((1568, FH), lambda i: (i, 0)),
        pl.BlockSpec((1568, 1), lambda i: (i, 0)),
        pl.BlockSpec((1568, 1), lambda i: (i, 0)),
        pl.BlockSpec((FH, HID), lambda i: (0, 0)),
        pl.BlockSpec((FH, HID), lambda i: (0, 0)),
        pl.BlockSpec((1, HID), lambda i: (0, 0)),
        pl.BlockSpec((1, HID), lambda i: (0, 0)),
    ],
    out_specs=pl.BlockSpec((1568, HID), lambda i: (i, 0)),
    out_shape=jax.ShapeDtypeStruct((NP, HID), f32),
)


def _tc_head_body(zp, cntp, w2, w3, b2, b3, out_o):
    acc = zp[0:G, :]
    for i in range(1, 32):
        acc = acc + zp[i * G:(i + 1) * G, :]
    cnt = jnp.sum(cntp[...], axis=1, keepdims=True)
    zc = acc * (1.0 / jnp.maximum(cnt, 1.0))
    w23 = jnp.dot(w2[...], w3[...], preferred_element_type=f32)
    b23 = jnp.dot(b2[...], w3[...], preferred_element_type=f32) + b3[...]
    out_o[...] = jnp.dot(zc, w23, preferred_element_type=f32) + b23


_tc_head = pl.pallas_call(
    _tc_head_body,
    out_shape=jax.ShapeDtypeStruct((G, C), f32),
)


def kernel(x, edge_index, batch, W1, b1, W2, b2, W3, b3):
    src = edge_index[0]
    dst = edge_index[1]
    iota_n = jnp.arange(N, dtype=i32)

    ones_hbm = jnp.ones((1568,), f32)
    zeros3200 = jnp.zeros((3200,), f32)
    zeros16 = jnp.zeros((3136, FH), f32)
    zeros3136 = jnp.zeros((3136,), f32)
    zeros64 = jnp.zeros((G, HID), f32)

    batcho = jnp.concatenate(
        [batch + NP, NP + 512 + (jnp.arange(NBP - N, dtype=i32) % 256)]
    )
    accflat = _k1_histograms(dst, batcho, ones_hbm, zeros3200)
    acc = accflat.reshape(2, DACC)
    deg0 = acc[0, :NP].reshape(NP, 1)
    deg1 = acc[1, :NP].reshape(NP, 1)
    cntp = jnp.concatenate(
        [acc[0, NP:NP + G, None], acc[1, NP:NP + G, None],
         jnp.zeros((G, 6), f32)], axis=1)

    xa = jnp.pad(x[:, :FH], ((0, NP - N), (0, 0)))
    xb = jnp.pad(x[:, FH:2 * FH], ((0, NP - N), (0, 0)))
    xc = jnp.pad(x[:, 2 * FH:], ((0, NP - N), (0, 0)))
    dinv2, va, vb, vc = _tc_scale(deg0, deg1, xa, xb, xc)

    sflat, s1 = _k3_segment_sum(va, vb, vc.reshape(NP), src, dst,
                                zeros16, zeros3136)
    sa = sflat[:NP]
    sb = sflat[NP:]

    u = _tc_hidden(sa, sb, s1.reshape(NP, 1), va, vb, vc, dinv2,
                   W1[:FH], W1[FH:2 * FH], W1[2 * FH:],
                   b1.reshape(1, HID))

    npad = E2P - E - N
    srcx = jnp.concatenate(
        [src, iota_n, N + (jnp.arange(npad, dtype=i32) % (NP - N))])
    dstx = jnp.concatenate([dst, iota_n, jnp.zeros((npad,), i32)])
    batchp = jnp.pad(batch, (0, NP - N))
    zflat = _k5_pool(u, srcx, dstx, dinv2.reshape(NP), batchp, zeros64)

    return _tc_head(zflat, cntp, W2, W3, b2.reshape(1, HID),
                    b3.reshape(1, C))


# speedup vs baseline: 20.9808x; 1.3987x over previous
import functools

import jax
import jax.numpy as jnp
from jax import lax
from jax.experimental import pallas as pl
from jax.experimental.pallas import tpu as pltpu
from jax.experimental.pallas import tpu_sc as plsc

N = 50000
E = 800000
FIN = 33
HID = 64
C = 6
G = 512

FH = 16
NP = 50176
DACC = 51200
NBP = 50176
EPT = E // 32
EPT3 = E // 16
ECH = 1000
E2P = 851968
EPT2 = E2P // 32
ECH2 = 512

f32 = jnp.float32
i32 = jnp.int32

_MESH = plsc.VectorSubcoreMesh(
    core_axis_name="c", subcore_axis_name="s", num_cores=2, num_subcores=16
)
_SC_PARAMS = pltpu.CompilerParams(use_tc_tiling_on_sc=False)


@functools.partial(
    pl.kernel,
    out_type=jax.ShapeDtypeStruct((2 * DACC,), f32),
    mesh=_MESH,
    compiler_params=_SC_PARAMS,
    scratch_types=[
        pltpu.VMEM((ECH,), i32),
        pltpu.VMEM((1568,), i32),
        pltpu.VMEM((1568,), f32),
        pltpu.VMEM_SHARED((DACC,), f32),
    ],
)
def _k1_histograms(dst_hbm, batcho_hbm, ones_hbm, zeros_hbm, out_hbm,
                   idx_e, idx_b, onesv, acc):
    cid = lax.axis_index("c")
    sid = lax.axis_index("s")
    tid = cid * 16 + sid
    pltpu.sync_copy(ones_hbm, onesv)
    pltpu.sync_copy(zeros_hbm, acc.at[pl.ds(sid * 3200, 3200)])
    plsc.subcore_barrier()

    @pl.loop(0, EPT // ECH)
    def _(i):
        base = tid * EPT + i * ECH
        pltpu.sync_copy(dst_hbm.at[pl.ds(base, ECH)], idx_e)
        pltpu.sync_copy(onesv.at[pl.ds(0, ECH)], acc.at[idx_e], add=True)

    pltpu.sync_copy(batcho_hbm.at[pl.ds(tid * 1568, 1568)], idx_b)
    pltpu.sync_copy(onesv, acc.at[idx_b], add=True)
    plsc.subcore_barrier()
    pltpu.sync_copy(
        acc.at[pl.ds(sid * 3200, 3200)],
        out_hbm.at[pl.ds(cid * DACC + sid * 3200, 3200)],
    )


@functools.partial(
    pl.kernel,
    out_type=(jax.ShapeDtypeStruct((2 * NP, FH), f32),
              jax.ShapeDtypeStruct((NP,), f32)),
    mesh=_MESH,
    compiler_params=_SC_PARAMS,
    scratch_types=[
        pltpu.VMEM((ECH,), i32),
        pltpu.VMEM((ECH,), i32),
        pltpu.VMEM((ECH, FH), f32),
        pltpu.VMEM((ECH,), f32),
        pltpu.VMEM_SHARED((NP, FH), f32),
        pltpu.VMEM_SHARED((NP,), f32),
    ],
)
def _k3_segment_sum(va_hbm, vb_hbm, vc_hbm, src_hbm, dst_hbm,
                    zeros_hbm, zeros1_hbm, out_hbm, out1_hbm,
                    idx_s, idx_d, rows, ele, acc, acc1):
    cid = lax.axis_index("c")
    sid = lax.axis_index("s")
    pltpu.sync_copy(zeros_hbm, acc.at[pl.ds(sid * 3136, 3136)])
    pltpu.sync_copy(zeros1_hbm, acc1.at[pl.ds(sid * 3136, 3136)])
    plsc.subcore_barrier()

    @pl.loop(0, EPT3 // ECH)
    def _(i):
        base = sid * EPT3 + i * ECH
        pltpu.sync_copy(src_hbm.at[pl.ds(base, ECH)], idx_s)
        pltpu.sync_copy(dst_hbm.at[pl.ds(base, ECH)], idx_d)

        @pl.when(cid == 0)
        def _():
            pltpu.sync_copy(va_hbm.at[idx_s], rows)

        @pl.when(cid == 1)
        def _():
            pltpu.sync_copy(vb_hbm.at[idx_s], rows)
            pltpu.sync_copy(vc_hbm.at[idx_s], ele)
            pltpu.sync_copy(ele, acc1.at[idx_d], add=True)

        pltpu.sync_copy(rows, acc.at[idx_d], add=True)

    plsc.subcore_barrier()
    pltpu.sync_copy(
        acc.at[pl.ds(sid * 3136, 3136)],
        out_hbm.at[pl.ds(cid * NP + sid * 3136, 3136)],
    )

    @pl.when(cid == 1)
    def _():
        pltpu.sync_copy(
            acc1.at[pl.ds(sid * 3136, 3136)],
            out1_hbm.at[pl.ds(sid * 3136, 3136)],
        )


@functools.partial(
    pl.kernel,
    out_type=jax.ShapeDtypeStruct((2 * G, HID), f32),
    mesh=_MESH,
    compiler_params=_SC_PARAMS,
    scratch_types=[
        pltpu.VMEM((ECH2,), i32),
        pltpu.VMEM((ECH2,), i32),
        pltpu.VMEM((ECH2,), i32),
        pltpu.VMEM((ECH2,), f32),
        pltpu.VMEM((ECH2, HID), f32),
        pltpu.VMEM((ECH2, HID), f32),
        pltpu.VMEM_SHARED((G, HID), f32),
    ],
)
def _k5_pool(u_hbm, srcx_hbm, dstx_hbm, dinv_hbm, batchp_hbm, zeros_hbm,
             out_hbm, idx_s, idx_d, gb, wb, rows, rows2, z):
    cid = lax.axis_index("c")
    sid = lax.axis_index("s")
    tid = cid * 16 + sid
    pltpu.sync_copy(zeros_hbm, z.at[pl.ds(sid * (G // 16), G // 16)])
    plsc.subcore_barrier()

    @pl.loop(0, EPT2 // ECH2)
    def _(i):
        base = tid * EPT2 + i * ECH2
        pltpu.sync_copy(srcx_hbm.at[pl.ds(base, ECH2)], idx_s)
        pltpu.sync_copy(dstx_hbm.at[pl.ds(base, ECH2)], idx_d)
        pltpu.sync_copy(u_hbm.at[idx_s], rows)
        pltpu.sync_copy(dinv_hbm.at[idx_d], wb)
        pltpu.sync_copy(batchp_hbm.at[idx_d], gb)

        @pl.loop(0, ECH2, step=16)
        def _(k):
            wv = wb[pl.ds(k, 16)]
            for l in range(16):
                w = wv[l]
                for j in range(HID // 16):
                    sl = pl.ds(j * 16, 16)
                    rows2[k + l, sl] = rows[k + l, sl] * w

        pltpu.sync_copy(rows2, z.at[gb], add=True)

    plsc.subcore_barrier()
    pltpu.sync_copy(
        z.at[pl.ds(sid * (G // 16), G // 16)],
        out_hbm.at[pl.ds(cid * G + sid * (G // 16), G // 16)],
    )


def _tc_scale_body(deg0, deg1, xa, xb, xc, dinv_o, va_o, vb_o, vc_o):
    d = deg0[...] + deg1[...] + 1.0
    dv = lax.rsqrt(jnp.maximum(d, 1.0))
    dinv_o[...] = dv
    va_o[...] = dv * xa[...]
    vb_o[...] = dv * xb[...]
    vc_o[...] = dv * xc[...]


_tc_scale = pl.pallas_call(
    _tc_scale_body,
    grid=(32,),
    in_specs=[
        pl.BlockSpec((1568, 1), lambda i: (i, 0)),
        pl.BlockSpec((1568, 1), lambda i: (i, 0)),
        pl.BlockSpec((1568, FH), lambda i: (i, 0)),
        pl.BlockSpec((1568, FH), lambda i: (i, 0)),
        pl.BlockSpec((1568, 1), lambda i: (i, 0)),
    ],
    out_specs=[
        pl.BlockSpec((1568, 1), lambda i: (i, 0)),
        pl.BlockSpec((1568, FH), lambda i: (i, 0)),
        pl.BlockSpec((1568, FH), lambda i: (i, 0)),
        pl.BlockSpec((1568, 1), lambda i: (i, 0)),
    ],
    out_shape=[
        jax.ShapeDtypeStruct((NP, 1), f32),
        jax.ShapeDtypeStruct((NP, FH), f32),
        jax.ShapeDtypeStruct((NP, FH), f32),
        jax.ShapeDtypeStruct((NP, 1), f32),
    ],
)


def _tc_hidden_body(sa, sb, sc, va, vb, vc, dinvb, w1a, w1b, w1c, b1, u_o):
    dv = dinvb[...]
    agg_a = dv * (sa[...] + va[...])
    agg_b = dv * (sb[...] + vb[...])
    agg_c = dv * (sc[...] + vc[...])
    h = (jnp.dot(agg_a, w1a[...], preferred_element_type=f32)
         + jnp.dot(agg_b, w1b[...], preferred_element_type=f32)
         + agg_c * w1c[...]
         + b1[...])
    h = jnp.maximum(h, 0.0)
    u = dv * h
    pid = pl.program_id(0)
    rid = pid * 1568 + lax.broadcasted_iota(i32, (1568, 1), 0)
    u_o[...] = jnp.where(rid < N, u, 0.0)


_tc_hidden = pl.pallas_call(
    _tc_hidden_body,
    grid=(32,),
    in_specs=[
        pl.BlockSpec((1568, FH), lambda i: (i, 0)),
        pl.BlockSpec((1568, FH), lambda i: (i, 0)),
        pl.BlockSpec((1568, 1), lambda i: (i, 0)),
        pl.BlockSpec((1568, FH), lambda i: (i, 0)),
        pl.BlockSpec((1568, FH), lambda i: (i, 0)),
        pl.BlockSpec((1568, 1), lambda i: (i, 0)),
        pl.BlockSpec((1568, 1), lambda i: (i, 0)),
        pl.BlockSpec((FH, HID), lambda i: (0, 0)),
        pl.BlockSpec((FH, HID), lambda i: (0, 0)),
        pl.BlockSpec((1, HID), lambda i: (0, 0)),
        pl.BlockSpec((1, HID), lambda i: (0, 0)),
    ],
    out_specs=pl.BlockSpec((1568, HID), lambda i: (i, 0)),
    out_shape=jax.ShapeDtypeStruct((NP, HID), f32),
)


def _tc_head_body(zp, cntp, w2, w3, b2, b3, out_o):
    acc = zp[0:G, :] + zp[G:2 * G, :]
    cnt = jnp.sum(cntp[...], axis=1, keepdims=True)
    zc = acc * (1.0 / jnp.maximum(cnt, 1.0))
    w23 = jnp.dot(w2[...], w3[...], preferred_element_type=f32)
    b23 = jnp.dot(b2[...], w3[...], preferred_element_type=f32) + b3[...]
    out_o[...] = jnp.dot(zc, w23, preferred_element_type=f32) + b23


_tc_head = pl.pallas_call(
    _tc_head_body,
    out_shape=jax.ShapeDtypeStruct((G, C), f32),
)


def kernel(x, edge_index, batch, W1, b1, W2, b2, W3, b3):
    src = edge_index[0]
    dst = edge_index[1]
    iota_n = jnp.arange(N, dtype=i32)

    ones_hbm = jnp.ones((1568,), f32)
    zeros3200 = jnp.zeros((3200,), f32)
    zeros16 = jnp.zeros((3136, FH), f32)
    zeros3136 = jnp.zeros((3136,), f32)
    zeros64 = jnp.zeros((G // 16, HID), f32)

    batcho = jnp.concatenate(
        [batch + NP, NP + 512 + (jnp.arange(NBP - N, dtype=i32) % 256)]
    )
    accflat = _k1_histograms(dst, batcho, ones_hbm, zeros3200)
    acc = accflat.reshape(2, DACC)
    deg0 = acc[0, :NP].reshape(NP, 1)
    deg1 = acc[1, :NP].reshape(NP, 1)
    cntp = jnp.concatenate(
        [acc[0, NP:NP + G, None], acc[1, NP:NP + G, None],
         jnp.zeros((G, 6), f32)], axis=1)

    xa = jnp.pad(x[:, :FH], ((0, NP - N), (0, 0)))
    xb = jnp.pad(x[:, FH:2 * FH], ((0, NP - N), (0, 0)))
    xc = jnp.pad(x[:, 2 * FH:], ((0, NP - N), (0, 0)))
    dinv2, va, vb, vc = _tc_scale(deg0, deg1, xa, xb, xc)

    sflat, s1 = _k3_segment_sum(va, vb, vc.reshape(NP), src, dst,
                                zeros16, zeros3136)
    sa = sflat[:NP]
    sb = sflat[NP:]

    u = _tc_hidden(sa, sb, s1.reshape(NP, 1), va, vb, vc, dinv2,
                   W1[:FH], W1[FH:2 * FH], W1[2 * FH:],
                   b1.reshape(1, HID))

    npad = E2P - E - N
    srcx = jnp.concatenate(
        [src, iota_n, N + (jnp.arange(npad, dtype=i32) % (NP - N))])
    dstx = jnp.concatenate([dst, iota_n, jnp.zeros((npad,), i32)])
    batchp = jnp.pad(batch, (0, NP - N))
    zflat = _k5_pool(u, srcx, dstx, dinv2.reshape(NP), batchp, zeros64)

    return _tc_head(zflat, cntp, W2, W3, b2.reshape(1, HID),
                    b3.reshape(1, C))


# speedup vs baseline: 21.8689x; 1.0423x over previous
import functools

import jax
import jax.numpy as jnp
from jax import lax
from jax.experimental import pallas as pl
from jax.experimental.pallas import tpu as pltpu
from jax.experimental.pallas import tpu_sc as plsc

N = 50000
E = 800000
FIN = 33
HID = 64
C = 6
G = 512

FH = 16
NP = 50176
DACC = 51200
NBP = 50176
EPT = E // 32
EPT3 = E // 16
ECH = 1000
E2P = 851968
EPT2 = E2P // 32
ECH2 = 512

f32 = jnp.float32
i32 = jnp.int32

_MESH = plsc.VectorSubcoreMesh(
    core_axis_name="c", subcore_axis_name="s", num_cores=2, num_subcores=16
)
_SC_PARAMS = pltpu.CompilerParams(use_tc_tiling_on_sc=False)


@functools.partial(
    pl.kernel,
    out_type=jax.ShapeDtypeStruct((2 * DACC,), f32),
    mesh=_MESH,
    compiler_params=_SC_PARAMS,
    scratch_types=[
        pltpu.VMEM((ECH,), i32),
        pltpu.VMEM((1568,), i32),
        pltpu.VMEM((1568,), f32),
        pltpu.VMEM_SHARED((DACC,), f32),
    ],
)
def _k1_histograms(dst_hbm, batcho_hbm, ones_hbm, zeros_hbm, out_hbm,
                   idx_e, idx_b, onesv, acc):
    cid = lax.axis_index("c")
    sid = lax.axis_index("s")
    tid = cid * 16 + sid
    pltpu.sync_copy(ones_hbm, onesv)
    pltpu.sync_copy(zeros_hbm, acc.at[pl.ds(sid * 3200, 3200)])
    plsc.subcore_barrier()

    @pl.loop(0, EPT // ECH)
    def _(i):
        base = tid * EPT + i * ECH
        pltpu.sync_copy(dst_hbm.at[pl.ds(base, ECH)], idx_e)
        pltpu.sync_copy(onesv.at[pl.ds(0, ECH)], acc.at[idx_e], add=True)

    pltpu.sync_copy(batcho_hbm.at[pl.ds(tid * 1568, 1568)], idx_b)
    pltpu.sync_copy(onesv, acc.at[idx_b], add=True)
    plsc.subcore_barrier()
    pltpu.sync_copy(
        acc.at[pl.ds(sid * 3200, 3200)],
        out_hbm.at[pl.ds(cid * DACC + sid * 3200, 3200)],
    )


@functools.partial(
    pl.kernel,
    out_type=(jax.ShapeDtypeStruct((2 * NP, FH), f32),
              jax.ShapeDtypeStruct((2 * NP,), f32)),
    mesh=_MESH,
    compiler_params=_SC_PARAMS,
    scratch_types=[
        pltpu.VMEM((ECH,), i32),
        pltpu.VMEM((ECH,), i32),
        pltpu.VMEM((ECH, FH), f32),
        pltpu.VMEM((ECH,), f32),
        pltpu.VMEM_SHARED((NP, FH), f32),
        pltpu.VMEM_SHARED((NP,), f32),
    ],
)
def _k3_segment_sum(va_hbm, vb_hbm, vc_hbm, src_hbm, dst_hbm,
                    zeros_hbm, zeros1_hbm, out_hbm, out1_hbm,
                    idx_s, idx_d, rows, ele, acc, acc1):
    cid = lax.axis_index("c")
    sid = lax.axis_index("s")
    pltpu.sync_copy(zeros_hbm, acc.at[pl.ds(sid * 3136, 3136)])
    pltpu.sync_copy(zeros1_hbm, acc1.at[pl.ds(sid * 3136, 3136)])
    plsc.subcore_barrier()

    @pl.loop(0, EPT3 // ECH)
    def _(i):
        base = sid * EPT3 + i * ECH
        pltpu.sync_copy(src_hbm.at[pl.ds(base, ECH)], idx_s)
        pltpu.sync_copy(dst_hbm.at[pl.ds(base, ECH)], idx_d)

        @pl.when(cid == 0)
        def _():
            pltpu.sync_copy(va_hbm.at[idx_s], rows)

        @pl.when(cid == 1)
        def _():
            pltpu.sync_copy(vb_hbm.at[idx_s], rows)

        @pl.when(cid == (i % 2))
        def _():
            pltpu.sync_copy(vc_hbm.at[idx_s], ele)
            pltpu.sync_copy(ele, acc1.at[idx_d], add=True)

        pltpu.sync_copy(rows, acc.at[idx_d], add=True)

    plsc.subcore_barrier()
    pltpu.sync_copy(
        acc.at[pl.ds(sid * 3136, 3136)],
        out_hbm.at[pl.ds(cid * NP + sid * 3136, 3136)],
    )

    pltpu.sync_copy(
        acc1.at[pl.ds(sid * 3136, 3136)],
        out1_hbm.at[pl.ds(cid * NP + sid * 3136, 3136)],
    )


@functools.partial(
    pl.kernel,
    out_type=jax.ShapeDtypeStruct((2 * G, HID), f32),
    mesh=_MESH,
    compiler_params=_SC_PARAMS,
    scratch_types=[
        pltpu.VMEM((ECH2,), i32),
        pltpu.VMEM((ECH2,), i32),
        pltpu.VMEM((ECH2,), i32),
        pltpu.VMEM((ECH2,), f32),
        pltpu.VMEM((ECH2, HID), f32),
        pltpu.VMEM((ECH2, HID), f32),
        pltpu.VMEM_SHARED((G, HID), f32),
    ],
)
def _k5_pool(u_hbm, srcx_hbm, dstx_hbm, dinv_hbm, batchp_hbm, zeros_hbm,
             out_hbm, idx_s, idx_d, gb, wb, rows, rows2, z):
    cid = lax.axis_index("c")
    sid = lax.axis_index("s")
    tid = cid * 16 + sid
    pltpu.sync_copy(zeros_hbm, z.at[pl.ds(sid * (G // 16), G // 16)])
    plsc.subcore_barrier()

    @pl.loop(0, EPT2 // ECH2)
    def _(i):
        base = tid * EPT2 + i * ECH2
        pltpu.sync_copy(srcx_hbm.at[pl.ds(base, ECH2)], idx_s)
        pltpu.sync_copy(dstx_hbm.at[pl.ds(base, ECH2)], idx_d)
        pltpu.sync_copy(u_hbm.at[idx_s], rows)
        pltpu.sync_copy(dinv_hbm.at[idx_d], wb)
        pltpu.sync_copy(batchp_hbm.at[idx_d], gb)

        @pl.loop(0, ECH2, step=16)
        def _(k):
            wv = wb[pl.ds(k, 16)]
            for l in range(16):
                w = wv[l]
                for j in range(HID // 16):
                    sl = pl.ds(j * 16, 16)
                    rows2[k + l, sl] = rows[k + l, sl] * w

        pltpu.sync_copy(rows2, z.at[gb], add=True)

    plsc.subcore_barrier()
    pltpu.sync_copy(
        z.at[pl.ds(sid * (G // 16), G // 16)],
        out_hbm.at[pl.ds(cid * G + sid * (G // 16), G // 16)],
    )


def _tc_scale_body(deg0, deg1, xa, xb, xc, dinv_o, va_o, vb_o, vc_o):
    d = deg0[...] + deg1[...] + 1.0
    dv = lax.rsqrt(jnp.maximum(d, 1.0))
    dinv_o[...] = dv
    va_o[...] = dv * xa[...]
    vb_o[...] = dv * xb[...]
    vc_o[...] = dv * xc[...]


_tc_scale = pl.pallas_call(
    _tc_scale_body,
    grid=(8,),
    in_specs=[
        pl.BlockSpec((6272, 1), lambda i: (i, 0)),
        pl.BlockSpec((6272, 1), lambda i: (i, 0)),
        pl.BlockSpec((6272, FH), lambda i: (i, 0)),
        pl.BlockSpec((6272, FH), lambda i: (i, 0)),
        pl.BlockSpec((6272, 1), lambda i: (i, 0)),
    ],
    out_specs=[
        pl.BlockSpec((6272, 1), lambda i: (i, 0)),
        pl.BlockSpec((6272, FH), lambda i: (i, 0)),
        pl.BlockSpec((6272, FH), lambda i: (i, 0)),
        pl.BlockSpec((6272, 1), lambda i: (i, 0)),
    ],
    out_shape=[
        jax.ShapeDtypeStruct((NP, 1), f32),
        jax.ShapeDtypeStruct((NP, FH), f32),
        jax.ShapeDtypeStruct((NP, FH), f32),
        jax.ShapeDtypeStruct((NP, 1), f32),
    ],
)


def _tc_hidden_body(sa, sb, sc, va, vb, vc, dinvb, w1a, w1b, w1c, b1, u_o):
    dv = dinvb[...]
    agg_a = dv * (sa[...] + va[...])
    agg_b = dv * (sb[...] + vb[...])
    agg_c = dv * (sc[...] + vc[...])
    h = (jnp.dot(agg_a, w1a[...], preferred_element_type=f32)
         + jnp.dot(agg_b, w1b[...], preferred_element_type=f32)
         + agg_c * w1c[...]
         + b1[...])
    h = jnp.maximum(h, 0.0)
    u = dv * h
    pid = pl.program_id(0)
    rid = pid * 6272 + lax.broadcasted_iota(i32, (6272, 1), 0)
    u_o[...] = jnp.where(rid < N, u, 0.0)


_tc_hidden = pl.pallas_call(
    _tc_hidden_body,
    grid=(8,),
    in_specs=[
        pl.BlockSpec((6272, FH), lambda i: (i, 0)),
        pl.BlockSpec((6272, FH), lambda i: (i, 0)),
        pl.BlockSpec((6272, 1), lambda i: (i, 0)),
        pl.BlockSpec((6272, FH), lambda i: (i, 0)),
        pl.BlockSpec((6272, FH), lambda i: (i, 0)),
        pl.BlockSpec((6272, 1), lambda i: (i, 0)),
        pl.BlockSpec((6272, 1), lambda i: (i, 0)),
        pl.BlockSpec((FH, HID), lambda i: (0, 0)),
        pl.BlockSpec((FH, HID), lambda i: (0, 0)),
        pl.BlockSpec((1, HID), lambda i: (0, 0)),
        pl.BlockSpec((1, HID), lambda i: (0, 0)),
    ],
    out_specs=pl.BlockSpec((6272, HID), lambda i: (i, 0)),
    out_shape=jax.ShapeDtypeStruct((NP, HID), f32),
)


def _tc_head_body(zp, cntp, w2, w3, b2, b3, out_o):
    acc = zp[0:G, :] + zp[G:2 * G, :]
    cnt = jnp.sum(cntp[...], axis=1, keepdims=True)
    zc = acc * (1.0 / jnp.maximum(cnt, 1.0))
    w23 = jnp.dot(w2[...], w3[...], preferred_element_type=f32)
    b23 = jnp.dot(b2[...], w3[...], preferred_element_type=f32) + b3[...]
    out_o[...] = jnp.dot(zc, w23, preferred_element_type=f32) + b23


_tc_head = pl.pallas_call(
    _tc_head_body,
    out_shape=jax.ShapeDtypeStruct((G, C), f32),
)


def kernel(x, edge_index, batch, W1, b1, W2, b2, W3, b3):
    src = edge_index[0]
    dst = edge_index[1]
    iota_n = jnp.arange(N, dtype=i32)

    ones_hbm = jnp.ones((1568,), f32)
    zeros3200 = jnp.zeros((3200,), f32)
    zeros16 = jnp.zeros((3136, FH), f32)
    zeros3136 = jnp.zeros((3136,), f32)
    zeros64 = jnp.zeros((G // 16, HID), f32)

    batcho = jnp.concatenate(
        [batch + NP, NP + 512 + (jnp.arange(NBP - N, dtype=i32) % 256)]
    )
    accflat = _k1_histograms(dst, batcho, ones_hbm, zeros3200)
    acc = accflat.reshape(2, DACC)
    deg0 = acc[0, :NP].reshape(NP, 1)
    deg1 = acc[1, :NP].reshape(NP, 1)
    cntp = jnp.concatenate(
        [acc[0, NP:NP + G, None], acc[1, NP:NP + G, None],
         jnp.zeros((G, 6), f32)], axis=1)

    xa = jnp.pad(x[:, :FH], ((0, NP - N), (0, 0)))
    xb = jnp.pad(x[:, FH:2 * FH], ((0, NP - N), (0, 0)))
    xc = jnp.pad(x[:, 2 * FH:], ((0, NP - N), (0, 0)))
    dinv2, va, vb, vc = _tc_scale(deg0, deg1, xa, xb, xc)

    sflat, s1 = _k3_segment_sum(va, vb, vc.reshape(NP), src, dst,
                                zeros16, zeros3136)
    sa = sflat[:NP]
    sb = sflat[NP:]
    sc_col = (s1[:NP] + s1[NP:]).reshape(NP, 1)

    u = _tc_hidden(sa, sb, sc_col, va, vb, vc, dinv2,
                   W1[:FH], W1[FH:2 * FH], W1[2 * FH:],
                   b1.reshape(1, HID))

    npad = E2P - E - N
    srcx = jnp.concatenate(
        [src, iota_n, N + (jnp.arange(npad, dtype=i32) % (NP - N))])
    dstx = jnp.concatenate([dst, iota_n, jnp.zeros((npad,), i32)])
    batchp = jnp.pad(batch, (0, NP - N))
    zflat = _k5_pool(u, srcx, dstx, dinv2.reshape(NP), batchp, zeros64)

    return _tc_head(zflat, cntp, W2, W3, b2.reshape(1, HID),
                    b3.reshape(1, C))


# speedup vs baseline: 27.7204x; 1.2676x over previous
import functools

import jax
import jax.numpy as jnp
from jax import lax
from jax.experimental import pallas as pl
from jax.experimental.pallas import tpu as pltpu
from jax.experimental.pallas import tpu_sc as plsc

N = 50000
E = 800000
FIN = 33
HID = 64
C = 6
G = 512

FH = 16
NP = 50176
DACC = 51200
NBP = 50176
EPT = E // 32
EPT3 = E // 16
ECH = 1000
E2P = 851968
EPT2 = E2P // 32
ECH2 = 512

f32 = jnp.float32
i32 = jnp.int32

_MESH = plsc.VectorSubcoreMesh(
    core_axis_name="c", subcore_axis_name="s", num_cores=2, num_subcores=16
)
_SC_PARAMS = pltpu.CompilerParams(use_tc_tiling_on_sc=False)


@functools.partial(
    pl.kernel,
    out_type=jax.ShapeDtypeStruct((2 * DACC,), f32),
    mesh=_MESH,
    compiler_params=_SC_PARAMS,
    scratch_types=[
        pltpu.VMEM((ECH,), i32),
        pltpu.VMEM((1568,), i32),
        pltpu.VMEM((1568,), f32),
        pltpu.VMEM_SHARED((DACC,), f32),
    ],
)
def _k1_histograms(dst_hbm, batcho_hbm, ones_hbm, zeros_hbm, out_hbm,
                   idx_e, idx_b, onesv, acc):
    cid = lax.axis_index("c")
    sid = lax.axis_index("s")
    tid = cid * 16 + sid
    pltpu.sync_copy(ones_hbm, onesv)
    pltpu.sync_copy(zeros_hbm, acc.at[pl.ds(sid * 3200, 3200)])
    plsc.subcore_barrier()

    @pl.loop(0, EPT // ECH)
    def _(i):
        base = tid * EPT + i * ECH
        pltpu.sync_copy(dst_hbm.at[pl.ds(base, ECH)], idx_e)
        pltpu.sync_copy(onesv.at[pl.ds(0, ECH)], acc.at[idx_e], add=True)

    pltpu.sync_copy(batcho_hbm.at[pl.ds(tid * 1568, 1568)], idx_b)
    pltpu.sync_copy(onesv, acc.at[idx_b], add=True)
    plsc.subcore_barrier()
    pltpu.sync_copy(
        acc.at[pl.ds(sid * 3200, 3200)],
        out_hbm.at[pl.ds(cid * DACC + sid * 3200, 3200)],
    )


@functools.partial(
    pl.kernel,
    out_type=(jax.ShapeDtypeStruct((2 * NP, FH), f32),
              jax.ShapeDtypeStruct((2 * NP,), f32)),
    mesh=_MESH,
    compiler_params=_SC_PARAMS,
    scratch_types=[
        pltpu.VMEM((ECH,), i32),
        pltpu.VMEM((ECH,), i32),
        pltpu.VMEM((ECH,), i32),
        pltpu.VMEM((ECH,), i32),
        pltpu.VMEM((ECH, FH), f32),
        pltpu.VMEM((ECH, FH), f32),
        pltpu.VMEM((ECH,), f32),
        pltpu.VMEM((ECH,), f32),
        pltpu.SemaphoreType.DMA,
        pltpu.SemaphoreType.DMA,
        pltpu.SemaphoreType.DMA,
        pltpu.SemaphoreType.DMA,
        pltpu.SemaphoreType.DMA,
        pltpu.SemaphoreType.DMA,
        pltpu.SemaphoreType.DMA,
        pltpu.SemaphoreType.DMA,
        pltpu.VMEM_SHARED((NP, FH), f32),
        pltpu.VMEM_SHARED((NP,), f32),
    ],
)
def _k3_segment_sum(va_hbm, vb_hbm, vc_hbm, src_hbm, dst_hbm,
                    zeros_hbm, zeros1_hbm, out_hbm, out1_hbm,
                    idxs0, idxs1, idxd0, idxd1, rows0, rows1, ele0, ele1,
                    sis0, sis1, sid0, sid1, sg0, sg1, se0, se1,
                    acc, acc1):
    cid = lax.axis_index("c")
    sid = lax.axis_index("s")
    pltpu.sync_copy(zeros_hbm, acc.at[pl.ds(sid * 3136, 3136)])
    pltpu.sync_copy(zeros1_hbm, acc1.at[pl.ds(sid * 3136, 3136)])
    plsc.subcore_barrier()

    @pl.loop(0, EPT3 // (2 * ECH))
    def _(p):
        base0 = sid * EPT3 + p * 2 * ECH
        base1 = base0 + ECH
        di0s = pltpu.async_copy(src_hbm.at[pl.ds(base0, ECH)], idxs0, sis0)
        di0d = pltpu.async_copy(dst_hbm.at[pl.ds(base0, ECH)], idxd0, sid0)
        di1s = pltpu.async_copy(src_hbm.at[pl.ds(base1, ECH)], idxs1, sis1)
        di1d = pltpu.async_copy(dst_hbm.at[pl.ds(base1, ECH)], idxd1, sid1)
        mine0 = cid == (p % 2)
        di0s.wait()

        @pl.when(cid == 0)
        def _():
            pltpu.async_copy(va_hbm.at[idxs0], rows0, sg0)

        @pl.when(cid == 1)
        def _():
            pltpu.async_copy(vb_hbm.at[idxs0], rows0, sg0)

        @pl.when(mine0)
        def _():
            pltpu.async_copy(vc_hbm.at[idxs0], ele0, se0)

        di1s.wait()

        @pl.when(cid == 0)
        def _():
            pltpu.async_copy(va_hbm.at[idxs1], rows1, sg1)

        @pl.when(cid == 1)
        def _():
            pltpu.async_copy(vb_hbm.at[idxs1], rows1, sg1)

        @pl.when(mine0)
        def _():
            pltpu.async_copy(vc_hbm.at[idxs1], ele1, se1)

        di0d.wait()
        pltpu.make_async_copy(va_hbm.at[idxs0], rows0, sg0).wait()
        pltpu.sync_copy(rows0, acc.at[idxd0], add=True)

        @pl.when(mine0)
        def _():
            pltpu.make_async_copy(vc_hbm.at[idxs0], ele0, se0).wait()
            pltpu.sync_copy(ele0, acc1.at[idxd0], add=True)

        di1d.wait()
        pltpu.make_async_copy(va_hbm.at[idxs1], rows1, sg1).wait()
        pltpu.sync_copy(rows1, acc.at[idxd1], add=True)

        @pl.when(mine0)
        def _():
            pltpu.make_async_copy(vc_hbm.at[idxs1], ele1, se1).wait()
            pltpu.sync_copy(ele1, acc1.at[idxd1], add=True)

    plsc.subcore_barrier()
    pltpu.sync_copy(
        acc.at[pl.ds(sid * 3136, 3136)],
        out_hbm.at[pl.ds(cid * NP + sid * 3136, 3136)],
    )

    pltpu.sync_copy(
        acc1.at[pl.ds(sid * 3136, 3136)],
        out1_hbm.at[pl.ds(cid * NP + sid * 3136, 3136)],
    )


@functools.partial(
    pl.kernel,
    out_type=jax.ShapeDtypeStruct((2 * G, HID), f32),
    mesh=_MESH,
    compiler_params=_SC_PARAMS,
    scratch_types=[
        pltpu.VMEM((EPT2 // 2,), i32),
        pltpu.VMEM((EPT2 // 2,), i32),
        pltpu.VMEM((ECH2,), i32),
        pltpu.VMEM((ECH2,), i32),
        pltpu.VMEM((ECH2,), f32),
        pltpu.VMEM((ECH2,), f32),
        pltpu.VMEM((ECH2, HID), f32),
        pltpu.VMEM((ECH2, HID), f32),
        pltpu.SemaphoreType.DMA,
        pltpu.SemaphoreType.DMA,
        pltpu.SemaphoreType.DMA,
        pltpu.SemaphoreType.DMA,
        pltpu.SemaphoreType.DMA,
        pltpu.SemaphoreType.DMA,
        pltpu.VMEM_SHARED((G, HID), f32),
    ],
)
def _k5_pool(u_hbm, srcx_hbm, dstx_hbm, dinv_hbm, batchp_hbm, zeros_hbm,
             out_hbm, sbig, dbig, gb0, gb1, wb0, wb1, rows0, rows1,
             s0r, s0w, s0g, s1r, s1w, s1g, z):
    cid = lax.axis_index("c")
    sid = lax.axis_index("s")
    tid = cid * 16 + sid
    pltpu.sync_copy(zeros_hbm, z.at[pl.ds(sid * (G // 16), G // 16)])
    plsc.subcore_barrier()

    def scale(rows, wb):
        @pl.loop(0, ECH2, step=16)
        def _(k):
            wv = wb[pl.ds(k, 16)]
            for l in range(16):
                w = wv[l]
                for j in range(HID // 16):
                    sl = pl.ds(j * 16, 16)
                    rows[k + l, sl] = rows[k + l, sl] * w

    @pl.loop(0, 2)
    def _(hf):
        hbase = tid * EPT2 + hf * (EPT2 // 2)
        pltpu.sync_copy(srcx_hbm.at[pl.ds(hbase, EPT2 // 2)], sbig)
        pltpu.sync_copy(dstx_hbm.at[pl.ds(hbase, EPT2 // 2)], dbig)

        @pl.loop(0, EPT2 // 2 // (2 * ECH2))
        def _(p):
            o0 = p * 2 * ECH2
            o1 = o0 + ECH2
            d0r = pltpu.async_copy(
                u_hbm.at[sbig.at[pl.ds(o0, ECH2)]], rows0, s0r)
            d0w = pltpu.async_copy(
                dinv_hbm.at[dbig.at[pl.ds(o0, ECH2)]], wb0, s0w)
            d0g = pltpu.async_copy(
                batchp_hbm.at[dbig.at[pl.ds(o0, ECH2)]], gb0, s0g)
            d1r = pltpu.async_copy(
                u_hbm.at[sbig.at[pl.ds(o1, ECH2)]], rows1, s1r)
            d1w = pltpu.async_copy(
                dinv_hbm.at[dbig.at[pl.ds(o1, ECH2)]], wb1, s1w)
            d1g = pltpu.async_copy(
                batchp_hbm.at[dbig.at[pl.ds(o1, ECH2)]], gb1, s1g)
            d0r.wait()
            d0w.wait()
            scale(rows0, wb0)
            d0g.wait()
            pltpu.sync_copy(rows0, z.at[gb0], add=True)
            d1r.wait()
            d1w.wait()
            scale(rows1, wb1)
            d1g.wait()
            pltpu.sync_copy(rows1, z.at[gb1], add=True)

    plsc.subcore_barrier()
    pltpu.sync_copy(
        z.at[pl.ds(sid * (G // 16), G // 16)],
        out_hbm.at[pl.ds(cid * G + sid * (G // 16), G // 16)],
    )


def _tc_scale_body(deg0, deg1, xa, xb, xc, dinv_o, va_o, vb_o, vc_o):
    d = deg0[...] + deg1[...] + 1.0
    dv = lax.rsqrt(jnp.maximum(d, 1.0))
    dinv_o[...] = dv
    va_o[...] = dv * xa[...]
    vb_o[...] = dv * xb[...]
    vc_o[...] = dv * xc[...]


_tc_scale = pl.pallas_call(
    _tc_scale_body,
    grid=(8,),
    in_specs=[
        pl.BlockSpec((6272, 1), lambda i: (i, 0)),
        pl.BlockSpec((6272, 1), lambda i: (i, 0)),
        pl.BlockSpec((6272, FH), lambda i: (i, 0)),
        pl.BlockSpec((6272, FH), lambda i: (i, 0)),
        pl.BlockSpec((6272, 1), lambda i: (i, 0)),
    ],
    out_specs=[
        pl.BlockSpec((6272, 1), lambda i: (i, 0)),
        pl.BlockSpec((6272, FH), lambda i: (i, 0)),
        pl.BlockSpec((6272, FH), lambda i: (i, 0)),
        pl.BlockSpec((6272, 1), lambda i: (i, 0)),
    ],
    out_shape=[
        jax.ShapeDtypeStruct((NP, 1), f32),
        jax.ShapeDtypeStruct((NP, FH), f32),
        jax.ShapeDtypeStruct((NP, FH), f32),
        jax.ShapeDtypeStruct((NP, 1), f32),
    ],
)


def _tc_hidden_body(sa, sb, sc, va, vb, vc, dinvb, w1a, w1b, w1c, b1, u_o):
    dv = dinvb[...]
    agg_a = dv * (sa[...] + va[...])
    agg_b = dv * (sb[...] + vb[...])
    agg_c = dv * (sc[...] + vc[...])
    h = (jnp.dot(agg_a, w1a[...], preferred_element_type=f32)
         + jnp.dot(agg_b, w1b[...], preferred_element_type=f32)
         + agg_c * w1c[...]
         + b1[...])
    h = jnp.maximum(h, 0.0)
    u = dv * h
    pid = pl.program_id(0)
    rid = pid * 6272 + lax.broadcasted_iota(i32, (6272, 1), 0)
    u_o[...] = jnp.where(rid < N, u, 0.0)


_tc_hidden = pl.pallas_call(
    _tc_hidden_body,
    grid=(8,),
    in_specs=[
        pl.BlockSpec((6272, FH), lambda i: (i, 0)),
        pl.BlockSpec((6272, FH), lambda i: (i, 0)),
        pl.BlockSpec((6272, 1), lambda i: (i, 0)),
        pl.BlockSpec((6272, FH), lambda i: (i, 0)),
        pl.BlockSpec((6272, FH), lambda i: (i, 0)),
        pl.BlockSpec((6272, 1), lambda i: (i, 0)),
        pl.BlockSpec((6272, 1), lambda i: (i, 0)),
        pl.BlockSpec((FH, HID), lambda i: (0, 0)),
        pl.BlockSpec((FH, HID), lambda i: (0, 0)),
        pl.BlockSpec((1, HID), lambda i: (0, 0)),
        pl.BlockSpec((1, HID), lambda i: (0, 0)),
    ],
    out_specs=pl.BlockSpec((6272, HID), lambda i: (i, 0)),
    out_shape=jax.ShapeDtypeStruct((NP, HID), f32),
)


def _tc_head_body(zp, cntp, w2, w3, b2, b3, out_o):
    acc = zp[0:G, :] + zp[G:2 * G, :]
    cnt = jnp.sum(cntp[...], axis=1, keepdims=True)
    zc = acc * (1.0 / jnp.maximum(cnt, 1.0))
    w23 = jnp.dot(w2[...], w3[...], preferred_element_type=f32)
    b23 = jnp.dot(b2[...], w3[...], preferred_element_type=f32) + b3[...]
    out_o[...] = jnp.dot(zc, w23, preferred_element_type=f32) + b23


_tc_head = pl.pallas_call(
    _tc_head_body,
    out_shape=jax.ShapeDtypeStruct((G, C), f32),
)


def kernel(x, edge_index, batch, W1, b1, W2, b2, W3, b3):
    src = edge_index[0]
    dst = edge_index[1]
    iota_n = jnp.arange(N, dtype=i32)

    ones_hbm = jnp.ones((1568,), f32)
    zeros3200 = jnp.zeros((3200,), f32)
    zeros16 = jnp.zeros((3136, FH), f32)
    zeros3136 = jnp.zeros((3136,), f32)
    zeros64 = jnp.zeros((G // 16, HID), f32)

    batcho = jnp.concatenate(
        [batch + NP, NP + 512 + (jnp.arange(NBP - N, dtype=i32) % 256)]
    )
    accflat = _k1_histograms(dst, batcho, ones_hbm, zeros3200)
    acc = accflat.reshape(2, DACC)
    deg0 = acc[0, :NP].reshape(NP, 1)
    deg1 = acc[1, :NP].reshape(NP, 1)
    cntp = jnp.concatenate(
        [acc[0, NP:NP + G, None], acc[1, NP:NP + G, None],
         jnp.zeros((G, 6), f32)], axis=1)

    xa = jnp.pad(x[:, :FH], ((0, NP - N), (0, 0)))
    xb = jnp.pad(x[:, FH:2 * FH], ((0, NP - N), (0, 0)))
    xc = jnp.pad(x[:, 2 * FH:], ((0, NP - N), (0, 0)))
    dinv2, va, vb, vc = _tc_scale(deg0, deg1, xa, xb, xc)

    sflat, s1 = _k3_segment_sum(va, vb, vc.reshape(NP), src, dst,
                                zeros16, zeros3136)
    sa = sflat[:NP]
    sb = sflat[NP:]
    sc_col = (s1[:NP] + s1[NP:]).reshape(NP, 1)

    u = _tc_hidden(sa, sb, sc_col, va, vb, vc, dinv2,
                   W1[:FH], W1[FH:2 * FH], W1[2 * FH:],
                   b1.reshape(1, HID))

    npad = E2P - E - N
    srcx = jnp.concatenate(
        [src, iota_n, N + (jnp.arange(npad, dtype=i32) % (NP - N))])
    dstx = jnp.concatenate([dst, iota_n, jnp.zeros((npad,), i32)])
    batchp = jnp.pad(batch, (0, NP - N))
    zflat = _k5_pool(u, srcx, dstx, dinv2.reshape(NP), batchp, zeros64)

    return _tc_head(zflat, cntp, W2, W3, b2.reshape(1, HID),
                    b3.reshape(1, C))


# speedup vs baseline: 28.3235x; 1.0218x over previous
import functools

import jax
import jax.numpy as jnp
from jax import lax
from jax.experimental import pallas as pl
from jax.experimental.pallas import tpu as pltpu
from jax.experimental.pallas import tpu_sc as plsc

N = 50000
E = 800000
FIN = 33
HID = 64
C = 6
G = 512

FH = 16
NP = 50176
DACC = 51200
NBP = 50176
EPT3 = E // 16
ECH = 1000
E2P = 851968
EPT2 = E2P // 32
ECH2 = 512

f32 = jnp.float32
i32 = jnp.int32

_MESH = plsc.VectorSubcoreMesh(
    core_axis_name="c", subcore_axis_name="s", num_cores=2, num_subcores=16
)
_SC_PARAMS = pltpu.CompilerParams(use_tc_tiling_on_sc=False)
_SC_PARAMS_NL = pltpu.CompilerParams(use_tc_tiling_on_sc=False,
                                     needs_layout_passes=False)


@functools.partial(
    pl.kernel,
    out_type=(jax.ShapeDtypeStruct((2 * NP, FH), f32),
              jax.ShapeDtypeStruct((2 * NP,), f32),
              jax.ShapeDtypeStruct((NP,), f32),
              jax.ShapeDtypeStruct((2048,), f32)),
    mesh=_MESH,
    compiler_params=_SC_PARAMS_NL,
    scratch_types=[
        pltpu.VMEM((ECH,), i32),
        pltpu.VMEM((ECH,), i32),
        pltpu.VMEM((ECH,), i32),
        pltpu.VMEM((ECH,), i32),
        pltpu.VMEM((1568,), i32),
        pltpu.VMEM((ECH, FH), f32),
        pltpu.VMEM((ECH, FH), f32),
        pltpu.VMEM((ECH,), f32),
        pltpu.VMEM((ECH,), f32),
        pltpu.VMEM((1568,), f32),
        pltpu.VMEM((3136,), f32),
        pltpu.VMEM((ECH,), f32),
        pltpu.VMEM((ECH,), f32),
        pltpu.SemaphoreType.DMA,
        pltpu.SemaphoreType.DMA,
        pltpu.SemaphoreType.DMA,
        pltpu.SemaphoreType.DMA,
        pltpu.SemaphoreType.DMA,
        pltpu.SemaphoreType.DMA,
        pltpu.SemaphoreType.DMA,
        pltpu.SemaphoreType.DMA,
        pltpu.SemaphoreType.DMA,
        pltpu.SemaphoreType.DMA,
        pltpu.VMEM_SHARED((NP, FH), f32),
        pltpu.VMEM_SHARED((NP,), f32),
        pltpu.VMEM_SHARED((DACC,), f32),
        pltpu.VMEM_SHARED((NP,), f32),
    ],
)
def _ka_fused(xa_hbm, xb_hbm, xc_hbm, src_hbm, dst_hbm, batcho_hbm,
              ones_hbm, zeros16_hbm, zeros1_hbm, zeros3200_hbm,
              outs_hbm, outs1_hbm, outdinv_hbm, outcnt_hbm,
              idxs0, idxs1, idxd0, idxd1, idxb, rows0, rows1, ele0, ele1,
              onesv, degbuf, wb0, wb1,
              sis0, sis1, sid0, sid1, sg0, sg1, se0, se1, sw0, sw1,
              accs, acc1, deghist, dinv_sp):
    cid = lax.axis_index("c")
    sid = lax.axis_index("s")

    pltpu.sync_copy(ones_hbm, onesv)
    pltpu.sync_copy(zeros3200_hbm, deghist.at[pl.ds(sid * 3200, 3200)])
    pltpu.sync_copy(zeros16_hbm, accs.at[pl.ds(sid * 3136, 3136)])
    pltpu.sync_copy(zeros1_hbm, acc1.at[pl.ds(sid * 3136, 3136)])
    plsc.subcore_barrier()

    @pl.loop(0, EPT3 // (2 * ECH))
    def _(p):
        base0 = sid * EPT3 + p * 2 * ECH
        di0 = pltpu.async_copy(dst_hbm.at[pl.ds(base0, ECH)], idxd0, sid0)
        di1 = pltpu.async_copy(dst_hbm.at[pl.ds(base0 + ECH, ECH)],
                               idxd1, sid1)
        di0.wait()
        pltpu.sync_copy(onesv.at[pl.ds(0, ECH)], deghist.at[idxd0], add=True)
        di1.wait()
        pltpu.sync_copy(onesv.at[pl.ds(0, ECH)], deghist.at[idxd1], add=True)

    bbase = cid * (NBP // 2) + sid * 1568
    pltpu.sync_copy(batcho_hbm.at[pl.ds(bbase, 1568)], idxb)
    pltpu.sync_copy(onesv, deghist.at[idxb], add=True)
    plsc.subcore_barrier()

    pltpu.sync_copy(deghist.at[pl.ds(sid * 3136, 3136)], degbuf)

    @pl.loop(0, 3136, step=16)
    def _(k):
        sl = pl.ds(k, 16)
        d = degbuf[sl] + 1.0
        yi = jnp.int32(0x5F3759DF) - lax.shift_right_logical(
            plsc.bitcast(d, i32), 1)
        y = plsc.bitcast(yi, f32)
        h = 0.5 * d
        y = y * (1.5 - h * y * y)
        y = y * (1.5 - h * y * y)
        y = y * (1.5 - h * y * y)
        degbuf[sl] = y

    pltpu.sync_copy(degbuf, dinv_sp.at[pl.ds(sid * 3136, 3136)])

    @pl.when(cid == 0)
    def _():
        pltpu.sync_copy(degbuf, outdinv_hbm.at[pl.ds(sid * 3136, 3136)])

    @pl.when(sid == 0)
    def _():
        pltpu.sync_copy(deghist.at[pl.ds(NP, 1024)],
                        outcnt_hbm.at[pl.ds(cid * 1024, 1024)])

    plsc.subcore_barrier()

    def scale_rows(rows, wb):
        @pl.loop(0, ECH, step=16)
        def _(k):
            wv = wb[pl.ds(k, 16)]
            for l in range(16):
                rows[k + l] = rows[k + l] * wv[l]

    def scale_ele(ele, wb):
        @pl.loop(0, ECH, step=16)
        def _(k):
            sl = pl.ds(k, 16)
            ele[sl] = ele[sl] * wb[sl]

    @pl.loop(0, EPT3 // (2 * ECH))
    def _(p):
        base0 = sid * EPT3 + p * 2 * ECH
        base1 = base0 + ECH
        di0s = pltpu.async_copy(src_hbm.at[pl.ds(base0, ECH)], idxs0, sis0)
        di0d = pltpu.async_copy(dst_hbm.at[pl.ds(base0, ECH)], idxd0, sid0)
        di1s = pltpu.async_copy(src_hbm.at[pl.ds(base1, ECH)], idxs1, sis1)
        di1d = pltpu.async_copy(dst_hbm.at[pl.ds(base1, ECH)], idxd1, sid1)
        mine = cid == (p % 2)
        di0s.wait()
        dw0 = pltpu.async_copy(dinv_sp.at[idxs0], wb0, sw0)

        @pl.when(cid == 0)
        def _():
            pltpu.async_copy(xa_hbm.at[idxs0], rows0, sg0)

        @pl.when(cid == 1)
        def _():
            pltpu.async_copy(xb_hbm.at[idxs0], rows0, sg0)

        @pl.when(mine)
        def _():
            pltpu.async_copy(xc_hbm.at[idxs0], ele0, se0)

        di1s.wait()
        dw1 = pltpu.async_copy(dinv_sp.at[idxs1], wb1, sw1)

        @pl.when(cid == 0)
        def _():
            pltpu.async_copy(xa_hbm.at[idxs1], rows1, sg1)

        @pl.when(cid == 1)
        def _():
            pltpu.async_copy(xb_hbm.at[idxs1], rows1, sg1)

        @pl.when(mine)
        def _():
            pltpu.async_copy(xc_hbm.at[idxs1], ele1, se1)

        pltpu.make_async_copy(xa_hbm.at[idxs0], rows0, sg0).wait()
        dw0.wait()
        scale_rows(rows0, wb0)
        di0d.wait()
        pltpu.sync_copy(rows0, accs.at[idxd0], add=True)

        @pl.when(mine)
        def _():
            pltpu.make_async_copy(xc_hbm.at[idxs0], ele0, se0).wait()
            scale_ele(ele0, wb0)
            pltpu.sync_copy(ele0, acc1.at[idxd0], add=True)

        pltpu.make_async_copy(xa_hbm.at[idxs1], rows1, sg1).wait()
        dw1.wait()
        scale_rows(rows1, wb1)
        di1d.wait()
        pltpu.sync_copy(rows1, accs.at[idxd1], add=True)

        @pl.when(mine)
        def _():
            pltpu.make_async_copy(xc_hbm.at[idxs1], ele1, se1).wait()
            scale_ele(ele1, wb1)
            pltpu.sync_copy(ele1, acc1.at[idxd1], add=True)

    plsc.subcore_barrier()
    pltpu.sync_copy(
        accs.at[pl.ds(sid * 3136, 3136)],
        outs_hbm.at[pl.ds(cid * NP + sid * 3136, 3136)],
    )
    pltpu.sync_copy(
        acc1.at[pl.ds(sid * 3136, 3136)],
        outs1_hbm.at[pl.ds(cid * NP + sid * 3136, 3136)],
    )


@functools.partial(
    pl.kernel,
    out_type=jax.ShapeDtypeStruct((2 * G, HID), f32),
    mesh=_MESH,
    compiler_params=_SC_PARAMS,
    scratch_types=[
        pltpu.VMEM((EPT2 // 2,), i32),
        pltpu.VMEM((EPT2 // 2,), i32),
        pltpu.VMEM((ECH2,), i32),
        pltpu.VMEM((ECH2,), i32),
        pltpu.VMEM((ECH2,), f32),
        pltpu.VMEM((ECH2,), f32),
        pltpu.VMEM((ECH2, HID), f32),
        pltpu.VMEM((ECH2, HID), f32),
        pltpu.SemaphoreType.DMA,
        pltpu.SemaphoreType.DMA,
        pltpu.SemaphoreType.DMA,
        pltpu.SemaphoreType.DMA,
        pltpu.SemaphoreType.DMA,
        pltpu.SemaphoreType.DMA,
        pltpu.VMEM_SHARED((G, HID), f32),
    ],
)
def _kc_pool(u_hbm, srcx_hbm, dstx_hbm, dinv_hbm, batchp_hbm, zeros_hbm,
             out_hbm, sbig, dbig, gb0, gb1, wb0, wb1, rows0, rows1,
             s0r, s0w, s0g, s1r, s1w, s1g, z):
    cid = lax.axis_index("c")
    sid = lax.axis_index("s")
    tid = cid * 16 + sid
    pltpu.sync_copy(zeros_hbm, z.at[pl.ds(sid * (G // 16), G // 16)])
    plsc.subcore_barrier()

    def scale(rows, wb):
        @pl.loop(0, ECH2, step=16)
        def _(k):
            wv = wb[pl.ds(k, 16)]
            for l in range(16):
                w = wv[l]
                for j in range(HID // 16):
                    sl = pl.ds(j * 16, 16)
                    rows[k + l, sl] = rows[k + l, sl] * w

    @pl.loop(0, 2)
    def _(hf):
        hbase = tid * EPT2 + hf * (EPT2 // 2)
        pltpu.sync_copy(srcx_hbm.at[pl.ds(hbase, EPT2 // 2)], sbig)
        pltpu.sync_copy(dstx_hbm.at[pl.ds(hbase, EPT2 // 2)], dbig)

        @pl.loop(0, EPT2 // 2 // (2 * ECH2))
        def _(p):
            o0 = p * 2 * ECH2
            o1 = o0 + ECH2
            d0r = pltpu.async_copy(
                u_hbm.at[sbig.at[pl.ds(o0, ECH2)]], rows0, s0r)
            d0w = pltpu.async_copy(
                dinv_hbm.at[dbig.at[pl.ds(o0, ECH2)]], wb0, s0w)
            d0g = pltpu.async_copy(
                batchp_hbm.at[dbig.at[pl.ds(o0, ECH2)]], gb0, s0g)
            d1r = pltpu.async_copy(
                u_hbm.at[sbig.at[pl.ds(o1, ECH2)]], rows1, s1r)
            d1w = pltpu.async_copy(
                dinv_hbm.at[dbig.at[pl.ds(o1, ECH2)]], wb1, s1w)
            d1g = pltpu.async_copy(
                batchp_hbm.at[dbig.at[pl.ds(o1, ECH2)]], gb1, s1g)
            d0r.wait()
            d0w.wait()
            scale(rows0, wb0)
            d0g.wait()
            pltpu.sync_copy(rows0, z.at[gb0], add=True)
            d1r.wait()
            d1w.wait()
            scale(rows1, wb1)
            d1g.wait()
            pltpu.sync_copy(rows1, z.at[gb1], add=True)

    plsc.subcore_barrier()
    pltpu.sync_copy(
        z.at[pl.ds(sid * (G // 16), G // 16)],
        out_hbm.at[pl.ds(cid * G + sid * (G // 16), G // 16)],
    )


def _tc_hidden_body(sa, sb, sc, xa, xb, xc, dinvb, w1a, w1b, w1c, b1, u_o):
    dv = dinvb[...]
    agg_a = dv * (sa[...] + dv * xa[...])
    agg_b = dv * (sb[...] + dv * xb[...])
    agg_c = dv * (sc[...] + dv * xc[...])
    h = (jnp.dot(agg_a, w1a[...], preferred_element_type=f32)
         + jnp.dot(agg_b, w1b[...], preferred_element_type=f32)
         + agg_c * w1c[...]
         + b1[...])
    h = jnp.maximum(h, 0.0)
    u = dv * h
    pid = pl.program_id(0)
    rid = pid * 6272 + lax.broadcasted_iota(i32, (6272, 1), 0)
    u_o[...] = jnp.where(rid < N, u, 0.0)


_tc_hidden = pl.pallas_call(
    _tc_hidden_body,
    grid=(8,),
    in_specs=[
        pl.BlockSpec((6272, FH), lambda i: (i, 0)),
        pl.BlockSpec((6272, FH), lambda i: (i, 0)),
        pl.BlockSpec((6272, 1), lambda i: (i, 0)),
        pl.BlockSpec((6272, FH), lambda i: (i, 0)),
        pl.BlockSpec((6272, FH), lambda i: (i, 0)),
        pl.BlockSpec((6272, 1), lambda i: (i, 0)),
        pl.BlockSpec((6272, 1), lambda i: (i, 0)),
        pl.BlockSpec((FH, HID), lambda i: (0, 0)),
        pl.BlockSpec((FH, HID), lambda i: (0, 0)),
        pl.BlockSpec((1, HID), lambda i: (0, 0)),
        pl.BlockSpec((1, HID), lambda i: (0, 0)),
    ],
    out_specs=pl.BlockSpec((6272, HID), lambda i: (i, 0)),
    out_shape=jax.ShapeDtypeStruct((NP, HID), f32),
)


def _tc_head_body(zp, cntp, w2, w3, b2, b3, out_o):
    acc = zp[0:G, :] + zp[G:2 * G, :]
    cnt = jnp.sum(cntp[...], axis=1, keepdims=True)
    zc = acc * (1.0 / jnp.maximum(cnt, 1.0))
    w23 = jnp.dot(w2[...], w3[...], preferred_element_type=f32)
    b23 = jnp.dot(b2[...], w3[...], preferred_element_type=f32) + b3[...]
    out_o[...] = jnp.dot(zc, w23, preferred_element_type=f32) + b23


_tc_head = pl.pallas_call(
    _tc_head_body,
    out_shape=jax.ShapeDtypeStruct((G, C), f32),
)


def kernel(x, edge_index, batch, W1, b1, W2, b2, W3, b3):
    src = edge_index[0]
    dst = edge_index[1]
    iota_n = jnp.arange(N, dtype=i32)

    ones_hbm = jnp.ones((1568,), f32)
    zeros3200 = jnp.zeros((3200,), f32)
    zeros16 = jnp.zeros((3136, FH), f32)
    zeros3136 = jnp.zeros((3136,), f32)
    zeros64 = jnp.zeros((G // 16, HID), f32)

    batcho = jnp.concatenate(
        [batch + NP, NP + 512 + (jnp.arange(NBP - N, dtype=i32) % 256)]
    )
    xa = jnp.pad(x[:, :FH], ((0, NP - N), (0, 0)))
    xb = jnp.pad(x[:, FH:2 * FH], ((0, NP - N), (0, 0)))
    xc = jnp.pad(x[:, 2 * FH], (0, NP - N))

    sflat, s1, dinv1, cntflat = _ka_fused(
        xa, xb, xc, src, dst, batcho, ones_hbm, zeros16, zeros3136,
        zeros3200)
    sa = sflat[:NP]
    sb = sflat[NP:]
    sc_col = (s1[:NP] + s1[NP:]).reshape(NP, 1)
    cntp = jnp.concatenate(
        [cntflat[:G, None], cntflat[1024:1024 + G, None],
         jnp.zeros((G, 6), f32)], axis=1)
    dinv2 = dinv1.reshape(NP, 1)

    u = _tc_hidden(sa, sb, sc_col, xa, xb, xc.reshape(NP, 1), dinv2,
                   W1[:FH], W1[FH:2 * FH], W1[2 * FH:],
                   b1.reshape(1, HID))

    npad = E2P - E - N
    srcx = jnp.concatenate(
        [src, iota_n, N + (jnp.arange(npad, dtype=i32) % (NP - N))])
    dstx = jnp.concatenate([dst, iota_n, jnp.zeros((npad,), i32)])
    batchp = jnp.pad(batch, (0, NP - N))
    zflat = _kc_pool(u, srcx, dstx, dinv1, batchp, zeros64)

    return _tc_head(zflat, cntp, W2, W3, b2.reshape(1, HID),
                    b3.reshape(1, C))


# speedup vs baseline: 29.9963x; 1.0591x over previous
import functools

import jax
import jax.numpy as jnp
from jax import lax
from jax.experimental import pallas as pl
from jax.experimental.pallas import tpu as pltpu
from jax.experimental.pallas import tpu_sc as plsc

N = 50000
E = 800000
FIN = 33
HID = 64
C = 6
G = 512

FH = 16
NP = 50176
DACC = 51200
NBP = 50176
EPT3 = E // 16
ECH = 1000
E2P = 851968
EPT2 = E2P // 32
ECH2 = 512

f32 = jnp.float32
i32 = jnp.int32

_MESH = plsc.VectorSubcoreMesh(
    core_axis_name="c", subcore_axis_name="s", num_cores=2, num_subcores=16
)
_SC_PARAMS = pltpu.CompilerParams(use_tc_tiling_on_sc=False)
_SC_PARAMS_NL = pltpu.CompilerParams(use_tc_tiling_on_sc=False,
                                     needs_layout_passes=False)


@functools.partial(
    pl.kernel,
    out_type=(jax.ShapeDtypeStruct((2 * NP, FH), f32),
              jax.ShapeDtypeStruct((2 * NP,), f32),
              jax.ShapeDtypeStruct((NP,), f32),
              jax.ShapeDtypeStruct((2048,), f32)),
    mesh=_MESH,
    compiler_params=_SC_PARAMS_NL,
    scratch_types=[
        pltpu.VMEM((ECH,), i32),
        pltpu.VMEM((ECH,), i32),
        pltpu.VMEM((ECH,), i32),
        pltpu.VMEM((ECH,), i32),
        pltpu.VMEM((1568,), i32),
        pltpu.VMEM((ECH, FH), f32),
        pltpu.VMEM((ECH, FH), f32),
        pltpu.VMEM((ECH,), f32),
        pltpu.VMEM((ECH,), f32),
        pltpu.VMEM((1568,), f32),
        pltpu.VMEM((3136,), f32),
        pltpu.VMEM((ECH,), f32),
        pltpu.VMEM((ECH,), f32),
        pltpu.SemaphoreType.DMA,
        pltpu.SemaphoreType.DMA,
        pltpu.SemaphoreType.DMA,
        pltpu.SemaphoreType.DMA,
        pltpu.SemaphoreType.DMA,
        pltpu.SemaphoreType.DMA,
        pltpu.SemaphoreType.DMA,
        pltpu.SemaphoreType.DMA,
        pltpu.SemaphoreType.DMA,
        pltpu.SemaphoreType.DMA,
        pltpu.SemaphoreType.DMA,
        pltpu.SemaphoreType.DMA,
        pltpu.VMEM_SHARED((NP, FH), f32),
        pltpu.VMEM_SHARED((NP,), f32),
        pltpu.VMEM_SHARED((DACC,), f32),
        pltpu.VMEM_SHARED((NP,), f32),
    ],
)
def _ka_fused(xa_hbm, xb_hbm, xc_hbm, src_hbm, dst_hbm, batcho_hbm,
              ones_hbm, zeros16_hbm, zeros1_hbm, zeros3200_hbm,
              outs_hbm, outs1_hbm, outdinv_hbm, outcnt_hbm,
              idxs0, idxs1, idxd0, idxd1, idxb, rows0, rows1, ele0, ele1,
              onesv, degbuf, wb0, wb1,
              sis0, sis1, sid0, sid1, sg0, sg1, se0, se1, sw0, sw1,
              sa0, sa1, accs, acc1, deghist, dinv_sp):
    cid = lax.axis_index("c")
    sid = lax.axis_index("s")

    pltpu.sync_copy(ones_hbm, onesv)
    pltpu.sync_copy(zeros3200_hbm, deghist.at[pl.ds(sid * 3200, 3200)])
    pltpu.sync_copy(zeros16_hbm, accs.at[pl.ds(sid * 3136, 3136)])
    pltpu.sync_copy(zeros1_hbm, acc1.at[pl.ds(sid * 3136, 3136)])
    plsc.subcore_barrier()

    @pl.loop(0, EPT3 // (2 * ECH))
    def _(p):
        base0 = sid * EPT3 + p * 2 * ECH

        @pl.when(p > 0)
        def _():
            pltpu.make_async_copy(
                onesv.at[pl.ds(0, ECH)], deghist.at[idxd0], sa0).wait()

        di0 = pltpu.async_copy(dst_hbm.at[pl.ds(base0, ECH)], idxd0, sid0)
        di0.wait()
        pltpu.async_copy(onesv.at[pl.ds(0, ECH)], deghist.at[idxd0], sa0,
                         add=True)

        @pl.when(p > 0)
        def _():
            pltpu.make_async_copy(
                onesv.at[pl.ds(0, ECH)], deghist.at[idxd1], sa1).wait()

        di1 = pltpu.async_copy(dst_hbm.at[pl.ds(base0 + ECH, ECH)],
                               idxd1, sid1)
        di1.wait()
        pltpu.async_copy(onesv.at[pl.ds(0, ECH)], deghist.at[idxd1], sa1,
                         add=True)

    pltpu.make_async_copy(
        onesv.at[pl.ds(0, ECH)], deghist.at[idxd0], sa0).wait()
    pltpu.make_async_copy(
        onesv.at[pl.ds(0, ECH)], deghist.at[idxd1], sa1).wait()
    bbase = cid * (NBP // 2) + sid * 1568
    pltpu.sync_copy(batcho_hbm.at[pl.ds(bbase, 1568)], idxb)
    pltpu.sync_copy(onesv, deghist.at[idxb], add=True)
    plsc.subcore_barrier()

    pltpu.sync_copy(deghist.at[pl.ds(sid * 3136, 3136)], degbuf)

    @pl.loop(0, 3136, step=16)
    def _(k):
        sl = pl.ds(k, 16)
        d = degbuf[sl] + 1.0
        yi = jnp.int32(0x5F3759DF) - lax.shift_right_logical(
            plsc.bitcast(d, i32), 1)
        y = plsc.bitcast(yi, f32)
        h = 0.5 * d
        y = y * (1.5 - h * y * y)
        y = y * (1.5 - h * y * y)
        y = y * (1.5 - h * y * y)
        degbuf[sl] = y

    pltpu.sync_copy(degbuf, dinv_sp.at[pl.ds(sid * 3136, 3136)])

    @pl.when(cid == 0)
    def _():
        pltpu.sync_copy(degbuf, outdinv_hbm.at[pl.ds(sid * 3136, 3136)])

    @pl.when(sid == 0)
    def _():
        pltpu.sync_copy(deghist.at[pl.ds(NP, 1024)],
                        outcnt_hbm.at[pl.ds(cid * 1024, 1024)])

    plsc.subcore_barrier()

    def scale_rows(rows, wb):
        @pl.loop(0, ECH, step=16)
        def _(k):
            wv = wb[pl.ds(k, 16)]
            for l in range(16):
                rows[k + l] = rows[k + l] * wv[l]

    def scale_ele(ele, wb):
        @pl.loop(0, ECH, step=16)
        def _(k):
            sl = pl.ds(k, 16)
            ele[sl] = ele[sl] * wb[sl]

    @pl.loop(0, EPT3 // (2 * ECH))
    def _(p):
        base0 = sid * EPT3 + p * 2 * ECH
        base1 = base0 + ECH

        @pl.when(p > 0)
        def _():
            pltpu.make_async_copy(rows0, accs.at[idxd0], sa0).wait()
            pltpu.make_async_copy(rows1, accs.at[idxd1], sa1).wait()

        di0s = pltpu.async_copy(src_hbm.at[pl.ds(base0, ECH)], idxs0, sis0)
        di0d = pltpu.async_copy(dst_hbm.at[pl.ds(base0, ECH)], idxd0, sid0)
        di1s = pltpu.async_copy(src_hbm.at[pl.ds(base1, ECH)], idxs1, sis1)
        di1d = pltpu.async_copy(dst_hbm.at[pl.ds(base1, ECH)], idxd1, sid1)
        mine = cid == (p % 2)
        di0s.wait()
        dw0 = pltpu.async_copy(dinv_sp.at[idxs0], wb0, sw0)

        @pl.when(cid == 0)
        def _():
            pltpu.async_copy(xa_hbm.at[idxs0], rows0, sg0)

        @pl.when(cid == 1)
        def _():
            pltpu.async_copy(xb_hbm.at[idxs0], rows0, sg0)

        @pl.when(mine)
        def _():
            pltpu.async_copy(xc_hbm.at[idxs0], ele0, se0)

        di1s.wait()
        dw1 = pltpu.async_copy(dinv_sp.at[idxs1], wb1, sw1)

        @pl.when(cid == 0)
        def _():
            pltpu.async_copy(xa_hbm.at[idxs1], rows1, sg1)

        @pl.when(cid == 1)
        def _():
            pltpu.async_copy(xb_hbm.at[idxs1], rows1, sg1)

        @pl.when(mine)
        def _():
            pltpu.async_copy(xc_hbm.at[idxs1], ele1, se1)

        pltpu.make_async_copy(xa_hbm.at[idxs0], rows0, sg0).wait()
        dw0.wait()
        scale_rows(rows0, wb0)
        di0d.wait()
        pltpu.async_copy(rows0, accs.at[idxd0], sa0, add=True)

        @pl.when(mine)
        def _():
            pltpu.make_async_copy(xc_hbm.at[idxs0], ele0, se0).wait()
            scale_ele(ele0, wb0)
            pltpu.sync_copy(ele0, acc1.at[idxd0], add=True)

        pltpu.make_async_copy(xa_hbm.at[idxs1], rows1, sg1).wait()
        dw1.wait()
        scale_rows(rows1, wb1)
        di1d.wait()
        pltpu.async_copy(rows1, accs.at[idxd1], sa1, add=True)

        @pl.when(mine)
        def _():
            pltpu.make_async_copy(xc_hbm.at[idxs1], ele1, se1).wait()
            scale_ele(ele1, wb1)
            pltpu.sync_copy(ele1, acc1.at[idxd1], add=True)

    pltpu.make_async_copy(rows0, accs.at[idxd0], sa0).wait()
    pltpu.make_async_copy(rows1, accs.at[idxd1], sa1).wait()
    plsc.subcore_barrier()
    pltpu.sync_copy(
        accs.at[pl.ds(sid * 3136, 3136)],
        outs_hbm.at[pl.ds(cid * NP + sid * 3136, 3136)],
    )
    pltpu.sync_copy(
        acc1.at[pl.ds(sid * 3136, 3136)],
        outs1_hbm.at[pl.ds(cid * NP + sid * 3136, 3136)],
    )


@functools.partial(
    pl.kernel,
    out_type=jax.ShapeDtypeStruct((2 * G, HID), f32),
    mesh=_MESH,
    compiler_params=_SC_PARAMS,
    scratch_types=[
        pltpu.VMEM((EPT2 // 2,), i32),
        pltpu.VMEM((EPT2 // 2,), i32),
        pltpu.VMEM((ECH2,), i32),
        pltpu.VMEM((ECH2,), i32),
        pltpu.VMEM((ECH2,), f32),
        pltpu.VMEM((ECH2,), f32),
        pltpu.VMEM((ECH2, HID), f32),
        pltpu.VMEM((ECH2, HID), f32),
        pltpu.SemaphoreType.DMA,
        pltpu.SemaphoreType.DMA,
        pltpu.SemaphoreType.DMA,
        pltpu.SemaphoreType.DMA,
        pltpu.SemaphoreType.DMA,
        pltpu.SemaphoreType.DMA,
        pltpu.SemaphoreType.DMA,
        pltpu.SemaphoreType.DMA,
        pltpu.VMEM_SHARED((G, HID), f32),
    ],
)
def _kc_pool(u_hbm, srcx_hbm, dstx_hbm, dinv_hbm, batchp_hbm, zeros_hbm,
             out_hbm, sbig, dbig, gb0, gb1, wb0, wb1, rows0, rows1,
             s0r, s0w, s0g, s1r, s1w, s1g, sz0, sz1, z):
    cid = lax.axis_index("c")
    sid = lax.axis_index("s")
    tid = cid * 16 + sid
    pltpu.sync_copy(zeros_hbm, z.at[pl.ds(sid * (G // 16), G // 16)])
    plsc.subcore_barrier()

    def scale(rows, wb):
        @pl.loop(0, ECH2, step=16)
        def _(k):
            wv = wb[pl.ds(k, 16)]
            for l in range(16):
                w = wv[l]
                for j in range(HID // 16):
                    sl = pl.ds(j * 16, 16)
                    rows[k + l, sl] = rows[k + l, sl] * w

    @pl.loop(0, 2)
    def _(hf):
        hbase = tid * EPT2 + hf * (EPT2 // 2)
        pltpu.sync_copy(srcx_hbm.at[pl.ds(hbase, EPT2 // 2)], sbig)
        pltpu.sync_copy(dstx_hbm.at[pl.ds(hbase, EPT2 // 2)], dbig)

        @pl.loop(0, EPT2 // 2 // (2 * ECH2))
        def _(p):
            o0 = p * 2 * ECH2
            o1 = o0 + ECH2

            @pl.when((hf > 0) | (p > 0))
            def _():
                pltpu.make_async_copy(rows0, z.at[gb0], sz0).wait()
                pltpu.make_async_copy(rows1, z.at[gb1], sz1).wait()

            d0r = pltpu.async_copy(
                u_hbm.at[sbig.at[pl.ds(o0, ECH2)]], rows0, s0r)
            d0w = pltpu.async_copy(
                dinv_hbm.at[dbig.at[pl.ds(o0, ECH2)]], wb0, s0w)
            d0g = pltpu.async_copy(
                batchp_hbm.at[dbig.at[pl.ds(o0, ECH2)]], gb0, s0g)
            d1r = pltpu.async_copy(
                u_hbm.at[sbig.at[pl.ds(o1, ECH2)]], rows1, s1r)
            d1w = pltpu.async_copy(
                dinv_hbm.at[dbig.at[pl.ds(o1, ECH2)]], wb1, s1w)
            d1g = pltpu.async_copy(
                batchp_hbm.at[dbig.at[pl.ds(o1, ECH2)]], gb1, s1g)
            d0r.wait()
            d0w.wait()
            scale(rows0, wb0)
            d0g.wait()
            pltpu.async_copy(rows0, z.at[gb0], sz0, add=True)
            d1r.wait()
            d1w.wait()
            scale(rows1, wb1)
            d1g.wait()
            pltpu.async_copy(rows1, z.at[gb1], sz1, add=True)

    pltpu.make_async_copy(rows0, z.at[gb0], sz0).wait()
    pltpu.make_async_copy(rows1, z.at[gb1], sz1).wait()
    plsc.subcore_barrier()
    pltpu.sync_copy(
        z.at[pl.ds(sid * (G // 16), G // 16)],
        out_hbm.at[pl.ds(cid * G + sid * (G // 16), G // 16)],
    )


def _tc_hidden_body(sa, sb, sc, xa, xb, xc, dinvb, w1a, w1b, w1c, b1, u_o):
    dv = dinvb[...]
    agg_a = dv * (sa[...] + dv * xa[...])
    agg_b = dv * (sb[...] + dv * xb[...])
    agg_c = dv * (sc[...] + dv * xc[...])
    h = (jnp.dot(agg_a, w1a[...], preferred_element_type=f32)
         + jnp.dot(agg_b, w1b[...], preferred_element_type=f32)
         + agg_c * w1c[...]
         + b1[...])
    h = jnp.maximum(h, 0.0)
    u = dv * h
    pid = pl.program_id(0)
    rid = pid * 6272 + lax.broadcasted_iota(i32, (6272, 1), 0)
    u_o[...] = jnp.where(rid < N, u, 0.0)


_tc_hidden = pl.pallas_call(
    _tc_hidden_body,
    grid=(8,),
    in_specs=[
        pl.BlockSpec((6272, FH), lambda i: (i, 0)),
        pl.BlockSpec((6272, FH), lambda i: (i, 0)),
        pl.BlockSpec((6272, 1), lambda i: (i, 0)),
        pl.BlockSpec((6272, FH), lambda i: (i, 0)),
        pl.BlockSpec((6272, FH), lambda i: (i, 0)),
        pl.BlockSpec((6272, 1), lambda i: (i, 0)),
        pl.BlockSpec((6272, 1), lambda i: (i, 0)),
        pl.BlockSpec((FH, HID), lambda i: (0, 0)),
        pl.BlockSpec((FH, HID), lambda i: (0, 0)),
        pl.BlockSpec((1, HID), lambda i: (0, 0)),
        pl.BlockSpec((1, HID), lambda i: (0, 0)),
    ],
    out_specs=pl.BlockSpec((6272, HID), lambda i: (i, 0)),
    out_shape=jax.ShapeDtypeStruct((NP, HID), f32),
)


def _tc_head_body(zp, cntp, w2, w3, b2, b3, out_o):
    acc = zp[0:G, :] + zp[G:2 * G, :]
    cnt = jnp.sum(cntp[...], axis=1, keepdims=True)
    zc = acc * (1.0 / jnp.maximum(cnt, 1.0))
    w23 = jnp.dot(w2[...], w3[...], preferred_element_type=f32)
    b23 = jnp.dot(b2[...], w3[...], preferred_element_type=f32) + b3[...]
    out_o[...] = jnp.dot(zc, w23, preferred_element_type=f32) + b23


_tc_head = pl.pallas_call(
    _tc_head_body,
    out_shape=jax.ShapeDtypeStruct((G, C), f32),
)


def kernel(x, edge_index, batch, W1, b1, W2, b2, W3, b3):
    src = edge_index[0]
    dst = edge_index[1]
    iota_n = jnp.arange(N, dtype=i32)

    ones_hbm = jnp.ones((1568,), f32)
    zeros3200 = jnp.zeros((3200,), f32)
    zeros16 = jnp.zeros((3136, FH), f32)
    zeros3136 = jnp.zeros((3136,), f32)
    zeros64 = jnp.zeros((G // 16, HID), f32)

    batcho = jnp.concatenate(
        [batch + NP, NP + 512 + (jnp.arange(NBP - N, dtype=i32) % 256)]
    )
    xa = jnp.pad(x[:, :FH], ((0, NP - N), (0, 0)))
    xb = jnp.pad(x[:, FH:2 * FH], ((0, NP - N), (0, 0)))
    xc = jnp.pad(x[:, 2 * FH], (0, NP - N))

    sflat, s1, dinv1, cntflat = _ka_fused(
        xa, xb, xc, src, dst, batcho, ones_hbm, zeros16, zeros3136,
        zeros3200)
    sa = sflat[:NP]
    sb = sflat[NP:]
    sc_col = (s1[:NP] + s1[NP:]).reshape(NP, 1)
    cntp = jnp.concatenate(
        [cntflat[:G, None], cntflat[1024:1024 + G, None],
         jnp.zeros((G, 6), f32)], axis=1)
    dinv2 = dinv1.reshape(NP, 1)

    u = _tc_hidden(sa, sb, sc_col, xa, xb, xc.reshape(NP, 1), dinv2,
                   W1[:FH], W1[FH:2 * FH], W1[2 * FH:],
                   b1.reshape(1, HID))

    npad = E2P - E - N
    srcx = jnp.concatenate(
        [src, iota_n, N + (jnp.arange(npad, dtype=i32) % (NP - N))])
    dstx = jnp.concatenate([dst, iota_n, jnp.zeros((npad,), i32)])
    batchp = jnp.pad(batch, (0, NP - N))
    zflat = _kc_pool(u, srcx, dstx, dinv1, batchp, zeros64)

    return _tc_head(zflat, cntp, W2, W3, b2.reshape(1, HID),
                    b3.reshape(1, C))
